# Initial kernel scaffold; baseline (speedup 1.0000x reference)
import functools

import jax
import jax.numpy as jnp
from jax import lax
from jax.experimental import pallas as pl
from jax.experimental.pallas import tpu as pltpu
from jax.experimental.pallas import tpu_sc as plsc

NC = 2
NS = 16
NW = NC * NS

CHUNK = 128


def _emb_call(n_rows, d, n_chunks):
    mesh = plsc.VectorSubcoreMesh(core_axis_name="c", subcore_axis_name="s")

    @functools.partial(
        pl.kernel,
        mesh=mesh,
        out_type=jax.ShapeDtypeStruct((n_rows, d), jnp.float32),
        scratch_types=[
            pltpu.VMEM((n_chunks, CHUNK), jnp.int32),
            pltpu.VMEM((CHUNK, d), jnp.float32),
            pltpu.SemaphoreType.DMA,
        ],
    )
    def emb(idx_hbm, table_hbm, out_hbm, idx_v, rows_v, sem):
        wid = lax.axis_index("s") * NC + lax.axis_index("c")
        base = wid * (n_chunks * CHUNK)
        pltpu.sync_copy(idx_hbm.at[wid], idx_v)

        def body(j, carry):
            pltpu.async_copy(table_hbm.at[idx_v.at[j]], rows_v, sem).wait()
            pltpu.sync_copy(rows_v, out_hbm.at[pl.ds(base + j * CHUNK, CHUNK)])
            return carry

        lax.fori_loop(0, n_chunks, body, 0)

    return emb


def kernel(token_ids, weight):
    b, s = token_ids.shape
    n, d = weight.shape
    total = b * s
    per_w = total // NW
    n_chunks = per_w // CHUNK
    idx = token_ids.reshape(NW, n_chunks, CHUNK).astype(jnp.int32)
    out = _emb_call(total, d, n_chunks)(idx, weight)
    return out.reshape(b, s, d)


# speedup vs baseline: 1.0236x; 1.0236x over previous
import functools

import jax
import jax.numpy as jnp
from jax import lax
from jax.experimental import pallas as pl
from jax.experimental.pallas import tpu as pltpu
from jax.experimental.pallas import tpu_sc as plsc

NC = 2
NS = 16
NW = NC * NS

CHUNK = 128


def _emb_call(n_rows, d, n_chunks):
    mesh = plsc.VectorSubcoreMesh(core_axis_name="c", subcore_axis_name="s")

    @functools.partial(
        pl.kernel,
        mesh=mesh,
        out_type=jax.ShapeDtypeStruct((n_rows, d), jnp.float32),
        scratch_types=[
            pltpu.VMEM((n_chunks, CHUNK), jnp.int32),
            pltpu.VMEM((CHUNK, d), jnp.float32),
            pltpu.SemaphoreType.DMA,
        ],
        compiler_params=pltpu.CompilerParams(use_tc_tiling_on_sc=False),
    )
    def emb(idx_hbm, table_hbm, out_hbm, idx_v, rows_v, sem):
        wid = lax.axis_index("s") * NC + lax.axis_index("c")
        base = wid * (n_chunks * CHUNK)
        pltpu.sync_copy(idx_hbm.at[wid], idx_v)

        def body(j, carry):
            pltpu.async_copy(table_hbm.at[idx_v.at[j]], rows_v, sem).wait()
            pltpu.sync_copy(rows_v, out_hbm.at[pl.ds(base + j * CHUNK, CHUNK)])
            return carry

        lax.fori_loop(0, n_chunks, body, 0)

    return emb


def kernel(token_ids, weight):
    b, s = token_ids.shape
    n, d = weight.shape
    total = b * s
    per_w = total // NW
    n_chunks = per_w // CHUNK
    idx = token_ids.reshape(NW, n_chunks, CHUNK).astype(jnp.int32)
    out = _emb_call(total, d, n_chunks)(idx, weight)
    return out.reshape(b, s, d)


# speedup vs baseline: 1.1134x; 1.0877x over previous
import functools

import jax
import jax.numpy as jnp
from jax import lax
from jax.experimental import pallas as pl
from jax.experimental.pallas import tpu as pltpu
from jax.experimental.pallas import tpu_sc as plsc

NC = 2
NS = 16
NW = NC * NS

CHUNK = 128
GATHERS_PER_SLAB = 8
SLAB = CHUNK * GATHERS_PER_SLAB


def _emb_call(n_rows, d, n_chunks):
    mesh = plsc.VectorSubcoreMesh(core_axis_name="c", subcore_axis_name="s")
    n_slabs = n_chunks // GATHERS_PER_SLAB

    @functools.partial(
        pl.kernel,
        mesh=mesh,
        out_type=jax.ShapeDtypeStruct((n_rows, d), jnp.float32),
        scratch_types=[
            pltpu.VMEM((n_chunks, CHUNK), jnp.int32),
            pltpu.VMEM((2, SLAB, d), jnp.float32),
            pltpu.SemaphoreType.DMA,
            pltpu.SemaphoreType.DMA,
            pltpu.SemaphoreType.DMA,
            pltpu.SemaphoreType.DMA,
        ],
        compiler_params=pltpu.CompilerParams(use_tc_tiling_on_sc=False),
    )
    def emb(idx_hbm, table_hbm, out_hbm, idx_v, rows_v, g_sem0, g_sem1,
            s_sem0, s_sem1):
        wid = lax.axis_index("s") * NC + lax.axis_index("c")
        base = wid * (n_chunks * CHUNK)
        pltpu.sync_copy(idx_hbm.at[wid], idx_v)

        def fire(g, b, g_sem):
            for j in range(GATHERS_PER_SLAB):
                pltpu.async_copy(
                    table_hbm.at[idx_v.at[g * GATHERS_PER_SLAB + j]],
                    rows_v.at[b].at[pl.ds(j * CHUNK, CHUNK)],
                    g_sem,
                )

        def drain(b, g_sem):
            pltpu.make_async_copy(
                table_hbm.at[pl.ds(0, SLAB)], rows_v.at[b], g_sem
            ).wait()

        def store(g, b, s_sem):
            pltpu.async_copy(
                rows_v.at[b], out_hbm.at[pl.ds(base + g * SLAB, SLAB)], s_sem
            )

        def wait_store(b, s_sem):
            pltpu.make_async_copy(
                table_hbm.at[pl.ds(0, SLAB)], rows_v.at[b], s_sem
            ).wait()

        fire(0, 0, g_sem0)

        def body(g, carry):
            b = lax.rem(g, 2)

            @pl.when(b == 0)
            def _even():
                @pl.when(g >= 1)
                def _():
                    wait_store(1, s_sem1)
                fire(g + 1, 1, g_sem1)
                drain(0, g_sem0)
                store(g, 0, s_sem0)

            @pl.when(b == 1)
            def _odd():
                wait_store(0, s_sem0)
                fire(g + 1, 0, g_sem0)
                drain(1, g_sem1)
                store(g, 1, s_sem1)

            return carry

        lax.fori_loop(0, n_slabs - 1, body, 0)

        g_last = n_slabs - 1
        b_last = g_last % 2
        g_sem_last = g_sem0 if b_last == 0 else g_sem1
        s_sem_last = s_sem0 if b_last == 0 else s_sem1
        drain(b_last, g_sem_last)
        if n_slabs > 1:
            b_prev = 1 - b_last
            wait_store(b_prev, s_sem0 if b_prev == 0 else s_sem1)
        store(g_last, b_last, s_sem_last)
        wait_store(b_last, s_sem_last)

    return emb


def kernel(token_ids, weight):
    b, s = token_ids.shape
    n, d = weight.shape
    total = b * s
    per_w = total // NW
    n_chunks = per_w // CHUNK
    idx = token_ids.reshape(NW, n_chunks, CHUNK).astype(jnp.int32)
    out = _emb_call(total, d, n_chunks)(idx, weight)
    return out.reshape(b, s, d)


# speedup vs baseline: 1.8072x; 1.6232x over previous
import functools

import jax
import jax.numpy as jnp
from jax import lax
from jax.experimental import pallas as pl
from jax.experimental.pallas import tpu as pltpu
from jax.experimental.pallas import tpu_sc as plsc

NC = 2
NS = 16
NW = NC * NS

RPS = 16


def _emb_call(n_tok, s, d):
    mesh = plsc.VectorSubcoreMesh(core_axis_name="c", subcore_axis_name="s")
    rows_per_w = n_tok // NW
    n_slabs = rows_per_w // RPS

    @functools.partial(
        pl.kernel,
        mesh=mesh,
        out_type=jax.ShapeDtypeStruct((n_tok, s, d), jnp.float32),
        scratch_types=[
            pltpu.VMEM((rows_per_w, s), jnp.int32),
            pltpu.VMEM((2, RPS, s, d), jnp.float32),
            pltpu.SemaphoreType.DMA,
            pltpu.SemaphoreType.DMA,
            pltpu.SemaphoreType.DMA,
            pltpu.SemaphoreType.DMA,
        ],
        compiler_params=pltpu.CompilerParams(use_tc_tiling_on_sc=False),
    )
    def emb(idx_hbm, table_hbm, out_hbm, idx_v, rows_v, g_sem0, g_sem1,
            s_sem0, s_sem1):
        wid = lax.axis_index("s") * NC + lax.axis_index("c")
        base = wid * rows_per_w
        pltpu.sync_copy(idx_hbm.at[pl.ds(base, rows_per_w)], idx_v)

        def fire(g, b, g_sem):
            for j in range(RPS):
                pltpu.async_copy(
                    table_hbm.at[idx_v.at[g * RPS + j]],
                    rows_v.at[b].at[j],
                    g_sem,
                )

        def drain(b, g_sem):
            pltpu.make_async_copy(
                out_hbm.at[pl.ds(0, RPS)], rows_v.at[b], g_sem
            ).wait()

        def store(g, b, s_sem):
            pltpu.async_copy(
                rows_v.at[b], out_hbm.at[pl.ds(base + g * RPS, RPS)], s_sem
            )

        def wait_store(b, s_sem):
            pltpu.make_async_copy(
                out_hbm.at[pl.ds(0, RPS)], rows_v.at[b], s_sem
            ).wait()

        fire(0, 0, g_sem0)

        def body(g, carry):
            b = lax.rem(g, 2)

            @pl.when(b == 0)
            def _even():
                @pl.when(g >= 1)
                def _():
                    wait_store(1, s_sem1)
                fire(g + 1, 1, g_sem1)
                drain(0, g_sem0)
                store(g, 0, s_sem0)

            @pl.when(b == 1)
            def _odd():
                wait_store(0, s_sem0)
                fire(g + 1, 0, g_sem0)
                drain(1, g_sem1)
                store(g, 1, s_sem1)

            return carry

        lax.fori_loop(0, n_slabs - 1, body, 0)

        g_last = n_slabs - 1
        b_last = g_last % 2
        g_sem_last = g_sem0 if b_last == 0 else g_sem1
        s_sem_last = s_sem0 if b_last == 0 else s_sem1
        drain(b_last, g_sem_last)
        if n_slabs > 1:
            b_prev = 1 - b_last
            wait_store(b_prev, s_sem0 if b_prev == 0 else s_sem1)
        store(g_last, b_last, s_sem_last)
        wait_store(b_last, s_sem_last)

    return emb


def kernel(token_ids, weight):
    n_tok, s = token_ids.shape
    n, d = weight.shape
    return _emb_call(n_tok, s, d)(token_ids, weight)
